# baseline (speedup 1.0000x reference)
import functools

import jax
import jax.numpy as jnp
from jax import lax
from jax.experimental import pallas as pl
from jax.experimental.pallas import tpu as pltpu
from jax.experimental.pallas import tpu_sc as plsc

EMB = 128
BATCH = 16384
TAB = 1000
TAB_PAD = 1024
NUM_CORES = 2
NUM_SUBCORES = 16
NW = NUM_CORES * NUM_SUBCORES
B_PER_W = BATCH // NW
CHUNK = 128
N_CHUNKS = B_PER_W // CHUNK
ROWS_PER_TILE = TAB_PAD // NUM_SUBCORES


@jax.jit
def _sc_gather(idx, table):
    mesh = plsc.VectorSubcoreMesh(core_axis_name="c", subcore_axis_name="s")

    @functools.partial(
        pl.kernel,
        mesh=mesh,
        out_type=jax.ShapeDtypeStruct((BATCH, EMB), jnp.float32),
        scratch_types=[
            pltpu.VMEM((B_PER_W,), jnp.int32),
            pltpu.VMEM((N_CHUNKS, CHUNK, EMB), jnp.float32),
            pltpu.VMEM_SHARED((TAB_PAD, EMB), jnp.float32),
            pltpu.SemaphoreType.DMA,
        ],
    )
    def k(table_hbm, idx_hbm, out_hbm, idx_v, rows_v, table_sh, sem):
        sid = lax.axis_index("s")
        wid = sid * NUM_CORES + lax.axis_index("c")
        base = wid * B_PER_W
        row0 = sid * ROWS_PER_TILE
        pltpu.sync_copy(
            table_hbm.at[pl.ds(row0, ROWS_PER_TILE)],
            table_sh.at[pl.ds(row0, ROWS_PER_TILE)],
        )
        pltpu.sync_copy(idx_hbm.at[pl.ds(base, B_PER_W)], idx_v)
        plsc.subcore_barrier()
        gathers = [
            pltpu.async_copy(
                table_sh.at[idx_v.at[pl.ds(j * CHUNK, CHUNK)]], rows_v.at[j], sem
            )
            for j in range(N_CHUNKS)
        ]
        for g in gathers:
            g.wait()
        stores = [
            pltpu.async_copy(
                rows_v.at[j], out_hbm.at[pl.ds(base + j * CHUNK, CHUNK)], sem
            )
            for j in range(N_CHUNKS)
        ]
        for s in stores:
            s.wait()

    return k(table, idx)


def kernel(t, pos_encoding):
    table_pad = jnp.pad(pos_encoding, ((0, TAB_PAD - TAB), (0, 0)))
    return _sc_gather(t.astype(jnp.int32), table_pad)


# speedup vs baseline: 1.0396x; 1.0396x over previous
import functools

import jax
import jax.numpy as jnp
from jax import lax
from jax.experimental import pallas as pl
from jax.experimental.pallas import tpu as pltpu
from jax.experimental.pallas import tpu_sc as plsc

EMB = 128
BATCH = 16384
TAB = 1000
TAB_PAD = 1024
NUM_CORES = 2
NUM_SUBCORES = 16
NW = NUM_CORES * NUM_SUBCORES
B_PER_W = BATCH // NW
CHUNK = 128
N_CHUNKS = B_PER_W // CHUNK
ROWS_PER_TILE = TAB_PAD // NUM_SUBCORES


@jax.jit
def _sc_gather(idx, table):
    mesh = plsc.VectorSubcoreMesh(core_axis_name="c", subcore_axis_name="s")

    @functools.partial(
        pl.kernel,
        mesh=mesh,
        out_type=jax.ShapeDtypeStruct((BATCH, EMB), jnp.float32),
        scratch_types=[
            pltpu.VMEM((B_PER_W,), jnp.int32),
            pltpu.VMEM((N_CHUNKS, CHUNK, EMB), jnp.float32),
            pltpu.VMEM_SHARED((TAB_PAD, EMB), jnp.float32),
        ]
        + [pltpu.SemaphoreType.DMA] * (N_CHUNKS + 1),
    )
    def k(table_hbm, idx_hbm, out_hbm, idx_v, rows_v, table_sh, *sems):
        gsems, ssem = sems[:N_CHUNKS], sems[N_CHUNKS]
        sid = lax.axis_index("s")
        wid = sid * NUM_CORES + lax.axis_index("c")
        base = wid * B_PER_W
        row0 = sid * ROWS_PER_TILE
        pltpu.sync_copy(
            table_hbm.at[pl.ds(row0, ROWS_PER_TILE)],
            table_sh.at[pl.ds(row0, ROWS_PER_TILE)],
        )
        pltpu.sync_copy(idx_hbm.at[pl.ds(base, B_PER_W)], idx_v)
        plsc.subcore_barrier()
        gathers = [
            pltpu.async_copy(
                table_sh.at[idx_v.at[pl.ds(j * CHUNK, CHUNK)]], rows_v.at[j],
                gsems[j],
            )
            for j in range(N_CHUNKS)
        ]
        stores = []
        for j in range(N_CHUNKS):
            gathers[j].wait()
            stores.append(
                pltpu.async_copy(
                    rows_v.at[j], out_hbm.at[pl.ds(base + j * CHUNK, CHUNK)], ssem
                )
            )
        for s in stores:
            s.wait()

    return k(table, idx)


def kernel(t, pos_encoding):
    table_pad = jnp.pad(pos_encoding, ((0, TAB_PAD - TAB), (0, 0)))
    return _sc_gather(t.astype(jnp.int32), table_pad)


# speedup vs baseline: 1.0605x; 1.0200x over previous
import functools

import jax
import jax.numpy as jnp
from jax import lax
from jax.experimental import pallas as pl
from jax.experimental.pallas import tpu as pltpu
from jax.experimental.pallas import tpu_sc as plsc

EMB = 128
BATCH = 16384
TAB = 1000
TAB_PAD = 1024
NUM_CORES = 2
NUM_SUBCORES = 16
NW = NUM_CORES * NUM_SUBCORES
B_PER_W = BATCH // NW
CHUNK = 64
N_CHUNKS = B_PER_W // CHUNK
ROWS_PER_TILE = TAB_PAD // NUM_SUBCORES


@jax.jit
def _sc_gather(idx, table):
    mesh = plsc.VectorSubcoreMesh(core_axis_name="c", subcore_axis_name="s")

    @functools.partial(
        pl.kernel,
        mesh=mesh,
        out_type=jax.ShapeDtypeStruct((BATCH, EMB), jnp.float32),
        scratch_types=[
            pltpu.VMEM((B_PER_W,), jnp.int32),
            pltpu.VMEM((N_CHUNKS, CHUNK, EMB), jnp.float32),
            pltpu.VMEM_SHARED((TAB_PAD, EMB), jnp.float32),
        ]
        + [pltpu.SemaphoreType.DMA] * (N_CHUNKS + 1),
    )
    def k(table_hbm, idx_hbm, out_hbm, idx_v, rows_v, table_sh, *sems):
        gsems, ssem = sems[:N_CHUNKS], sems[N_CHUNKS]
        sid = lax.axis_index("s")
        wid = sid * NUM_CORES + lax.axis_index("c")
        base = wid * B_PER_W
        row0 = sid * ROWS_PER_TILE
        tcopy = pltpu.async_copy(
            table_hbm.at[pl.ds(row0, ROWS_PER_TILE)],
            table_sh.at[pl.ds(row0, ROWS_PER_TILE)],
            ssem,
        )
        icopy = pltpu.async_copy(idx_hbm.at[pl.ds(base, B_PER_W)], idx_v, ssem)
        tcopy.wait()
        icopy.wait()
        plsc.subcore_barrier()
        gathers = [
            pltpu.async_copy(
                table_sh.at[idx_v.at[pl.ds(j * CHUNK, CHUNK)]], rows_v.at[j],
                gsems[j],
            )
            for j in range(N_CHUNKS)
        ]
        stores = []
        for j in range(N_CHUNKS):
            gathers[j].wait()
            stores.append(
                pltpu.async_copy(
                    rows_v.at[j], out_hbm.at[pl.ds(base + j * CHUNK, CHUNK)], ssem
                )
            )
        for s in stores:
            s.wait()

    return k(table, idx)


def kernel(t, pos_encoding):
    table_pad = jnp.pad(pos_encoding, ((0, TAB_PAD - TAB), (0, 0)))
    return _sc_gather(t.astype(jnp.int32), table_pad)


# speedup vs baseline: 1.0680x; 1.0071x over previous
import functools

import jax
import jax.numpy as jnp
from jax import lax
from jax.experimental import pallas as pl
from jax.experimental.pallas import tpu as pltpu
from jax.experimental.pallas import tpu_sc as plsc

EMB = 128
BATCH = 16384
TAB = 1000
TAB_PAD = 1024
NUM_CORES = 2
NUM_SUBCORES = 16
NW = NUM_CORES * NUM_SUBCORES
B_PER_W = BATCH // NW
CHUNK = 32
N_CHUNKS = B_PER_W // CHUNK
ROWS_PER_TILE = TAB_PAD // NUM_SUBCORES


@jax.jit
def _sc_gather(idx, table):
    mesh = plsc.VectorSubcoreMesh(core_axis_name="c", subcore_axis_name="s")

    @functools.partial(
        pl.kernel,
        mesh=mesh,
        out_type=jax.ShapeDtypeStruct((BATCH, EMB), jnp.float32),
        scratch_types=[
            pltpu.VMEM((B_PER_W,), jnp.int32),
            pltpu.VMEM((N_CHUNKS, CHUNK, EMB), jnp.float32),
            pltpu.VMEM_SHARED((TAB_PAD, EMB), jnp.float32),
        ]
        + [pltpu.SemaphoreType.DMA] * (N_CHUNKS + 1),
    )
    def k(table_hbm, idx_hbm, out_hbm, idx_v, rows_v, table_sh, *sems):
        gsems, ssem = sems[:N_CHUNKS], sems[N_CHUNKS]
        sid = lax.axis_index("s")
        wid = sid * NUM_CORES + lax.axis_index("c")
        base = wid * B_PER_W
        row0 = sid * ROWS_PER_TILE
        tcopy = pltpu.async_copy(
            table_hbm.at[pl.ds(row0, ROWS_PER_TILE)],
            table_sh.at[pl.ds(row0, ROWS_PER_TILE)],
            ssem,
        )
        icopy = pltpu.async_copy(idx_hbm.at[pl.ds(base, B_PER_W)], idx_v, ssem)
        tcopy.wait()
        icopy.wait()
        plsc.subcore_barrier()
        gathers = [
            pltpu.async_copy(
                table_sh.at[idx_v.at[pl.ds(j * CHUNK, CHUNK)]], rows_v.at[j],
                gsems[j],
            )
            for j in range(N_CHUNKS)
        ]
        stores = []
        for j in range(N_CHUNKS):
            gathers[j].wait()
            stores.append(
                pltpu.async_copy(
                    rows_v.at[j], out_hbm.at[pl.ds(base + j * CHUNK, CHUNK)], ssem
                )
            )
        for s in stores:
            s.wait()

    return k(table, idx)


def kernel(t, pos_encoding):
    table_pad = jnp.pad(pos_encoding, ((0, TAB_PAD - TAB), (0, 0)))
    return _sc_gather(t.astype(jnp.int32), table_pad)
